# Initial kernel scaffold; baseline (speedup 1.0000x reference)
import jax
import jax.numpy as jnp
from jax.experimental import pallas as pl


def kernel(adj_est, x, W1_rel, b1_rel, W1_root, W2_rel, b2_rel, W2_root):
    raise NotImplementedError("write your pallas kernel here")


# speedup vs baseline: 7.2304x; 7.2304x over previous
import functools

import jax
import jax.numpy as jnp
from jax import lax
from jax.experimental import pallas as pl
from jax.experimental.pallas import tpu as pltpu
from jax.experimental.pallas import tpu_sc as plsc

_N = 10000
_DI = 128
_E = 320000
_NSC = 2
_NSUB = 16
_NTILES = _NSC * _NSUB
_CH = 128
_NCHUNK = 79
_EPT = _NCHUNK * _CH
_EPAD = _NTILES * _EPT
_NROWS = 10112
_RPT = _NROWS // _NSUB


def _make_segsum(d):
  mesh = plsc.VectorSubcoreMesh(core_axis_name="c", subcore_axis_name="s")

  @functools.partial(
      pl.kernel,
      mesh=mesh,
      compiler_params=pltpu.CompilerParams(use_tc_tiling_on_sc=False),
      out_type=jax.ShapeDtypeStruct((_NSC, _NROWS, d), jnp.float32),
      scratch_types=[
          pltpu.VMEM((_NCHUNK, _CH), jnp.int32),
          pltpu.VMEM((_NCHUNK, _CH), jnp.int32),
          pltpu.VMEM((_CH, d), jnp.float32),
          pltpu.VMEM_SHARED((_NROWS, d), jnp.float32),
          pltpu.SemaphoreType.DMA,
      ],
  )
  def segsum(table, srcg, dstg, init, out, sidx, didx, rows, acc, sem):
    c = lax.axis_index("c")
    s = lax.axis_index("s")
    wid = c * _NSUB + s
    r0 = s * _RPT
    pltpu.sync_copy(init.at[c, pl.ds(r0, _RPT)], acc.at[pl.ds(r0, _RPT)])
    pltpu.sync_copy(srcg.at[wid], sidx)
    pltpu.sync_copy(dstg.at[wid], didx)
    plsc.subcore_barrier()

    def body(j, carry):
      pltpu.async_copy(table.at[sidx.at[j]], rows, sem).wait()
      pltpu.sync_copy(rows, acc.at[didx.at[j]], add=True)
      return carry

    lax.fori_loop(0, _NCHUNK, body, 0)
    plsc.subcore_barrier()
    pltpu.sync_copy(acc.at[pl.ds(r0, _RPT)], out.at[c, pl.ds(r0, _RPT)])

  return segsum


_SEGSUM128 = _make_segsum(_DI)
_SEGSUM16 = _make_segsum(16)

_BM = 1000


def _dense_mid(parts, xf, w1r, b1, w1o, w2r, w2o, b2):
  def body(p0, p1, xb, w1r_r, b1_r, w1o_r, w2r_r, w2o_r, b2_r, p2_o, r2_o):
    agg = p0[0] + p1[0]
    h = jnp.dot(agg, w1r_r[...], preferred_element_type=jnp.float32)
    h += b1_r[...]
    h += jnp.dot(xb[...], w1o_r[...], preferred_element_type=jnp.float32)
    h = jnp.maximum(h, 0.0)
    p2_o[...] = jnp.dot(h, w2r_r[...], preferred_element_type=jnp.float32)
    r2_o[...] = jnp.dot(h, w2o_r[...], preferred_element_type=jnp.float32) + b2_r[...]

  return pl.pallas_call(
      body,
      grid=(_N // _BM,),
      in_specs=[
          pl.BlockSpec((1, _BM, _DI), lambda i: (0, i, 0)),
          pl.BlockSpec((1, _BM, _DI), lambda i: (1, i, 0)),
          pl.BlockSpec((_BM, _DI), lambda i: (i, 0)),
          pl.BlockSpec((_DI, _DI), lambda i: (0, 0)),
          pl.BlockSpec((1, _DI), lambda i: (0, 0)),
          pl.BlockSpec((_DI, _DI), lambda i: (0, 0)),
          pl.BlockSpec((_DI, 16), lambda i: (0, 0)),
          pl.BlockSpec((_DI, 16), lambda i: (0, 0)),
          pl.BlockSpec((1, 16), lambda i: (0, 0)),
      ],
      out_specs=[
          pl.BlockSpec((_BM, 16), lambda i: (i, 0)),
          pl.BlockSpec((_BM, 16), lambda i: (i, 0)),
      ],
      out_shape=[
          jax.ShapeDtypeStruct((_N, 16), jnp.float32),
          jax.ShapeDtypeStruct((_N, 16), jnp.float32),
      ],
  )(parts, parts, xf, w1r, b1, w1o, w2r, w2o, b2)


def _final_add(parts2):
  def body(q0, q1, o):
    o[...] = q0[0] + q1[0]

  return pl.pallas_call(
      body,
      grid=(_N // _BM,),
      in_specs=[
          pl.BlockSpec((1, _BM, 16), lambda i: (0, i, 0)),
          pl.BlockSpec((1, _BM, 16), lambda i: (1, i, 0)),
      ],
      out_specs=pl.BlockSpec((_BM, 16), lambda i: (i, 0)),
      out_shape=jax.ShapeDtypeStruct((_N, 16), jnp.float32),
  )(parts2, parts2)


def kernel(adj_est, x, W1_rel, b1_rel, W1_root, W2_rel, b2_rel, W2_root):
  xf = x.reshape(_N, _DI)
  src = jnp.pad(adj_est[0], (0, _EPAD - _E)).reshape(_NTILES, _NCHUNK, _CH)
  dst = jnp.pad(adj_est[1], (0, _EPAD - _E), constant_values=_N).reshape(
      _NTILES, _NCHUNK, _CH)

  init1 = jnp.zeros((_NSC, _NROWS, _DI), jnp.float32)
  parts1 = _SEGSUM128(xf, src, dst, init1)

  w2r = jnp.pad(W2_rel, ((0, 0), (0, 16 - W2_rel.shape[1])))
  w2o = jnp.pad(W2_root, ((0, 0), (0, 16 - W2_root.shape[1])))
  b2 = jnp.pad(b2_rel, (0, 16 - b2_rel.shape[0])).reshape(1, 16)
  p2, r2 = _dense_mid(parts1, xf, W1_rel, b1_rel.reshape(1, _DI), W1_root,
                      w2r, w2o, b2)

  init2 = jnp.zeros((_NSC, _NROWS, 16), jnp.float32).at[0, :_N, :].set(r2)
  parts2 = _SEGSUM16(p2, src, dst, init2)

  out16 = _final_add(parts2)
  return out16[:, :7].reshape(1, _N, 7)


# speedup vs baseline: 9.7428x; 1.3475x over previous
import functools

import jax
import jax.numpy as jnp
from jax import lax
from jax.experimental import pallas as pl
from jax.experimental.pallas import tpu as pltpu
from jax.experimental.pallas import tpu_sc as plsc

_N = 10000
_DI = 128
_E = 320000
_NSC = 2
_NSUB = 16
_NTILES = _NSC * _NSUB
_NROWS = 10016
_RPT = _NROWS // _NSUB
_CH1, _NCHUNK1 = 56, 180
_CH2, _NCHUNK2 = 128, 80
_NB = 4


def _make_segsum(d, ch, nchunk):
  mesh = plsc.VectorSubcoreMesh(core_axis_name="c", subcore_axis_name="s")

  @functools.partial(
      pl.kernel,
      mesh=mesh,
      compiler_params=pltpu.CompilerParams(use_tc_tiling_on_sc=False),
      out_type=jax.ShapeDtypeStruct((_NSC, _NROWS, d), jnp.float32),
      scratch_types=[
          pltpu.VMEM((nchunk, ch), jnp.int32),
          pltpu.VMEM((nchunk, ch), jnp.int32),
          [pltpu.VMEM((ch, d), jnp.float32)] * _NB,
          pltpu.VMEM_SHARED((_NROWS, d), jnp.float32),
          [pltpu.SemaphoreType.DMA] * _NB,
          [pltpu.SemaphoreType.DMA] * _NB,
      ],
  )
  def segsum(table, srcg, dstg, init, out, sidx, didx, rows, acc, gsem, ssem):
    c = lax.axis_index("c")
    s = lax.axis_index("s")
    wid = c * _NSUB + s
    r0 = s * _RPT
    pltpu.sync_copy(srcg.at[wid], sidx)
    pltpu.sync_copy(dstg.at[wid], didx)
    for b in range(2):
      pltpu.async_copy(table.at[sidx.at[b]], rows[b], gsem[b])
    pltpu.sync_copy(init.at[c, pl.ds(r0, _RPT)], acc.at[pl.ds(r0, _RPT)])
    plsc.subcore_barrier()

    def grp(k, carry):
      j0 = _NB * k
      for b in range(_NB):
        j = j0 + b
        pltpu.make_async_copy(table.at[sidx.at[j]], rows[b], gsem[b]).wait()
        pltpu.async_copy(rows[b], acc.at[didx.at[j]], ssem[b], add=True)
        b2 = (b + 2) % _NB

        @pl.when(j >= 2)
        def _():
          pltpu.make_async_copy(
              rows[b2], acc.at[didx.at[j - 2]], ssem[b2]).wait()

        @pl.when(j + 2 < nchunk)
        def _():
          pltpu.async_copy(table.at[sidx.at[j + 2]], rows[b2], gsem[b2])
      return carry

    lax.fori_loop(0, nchunk // _NB, grp, 0)
    for j in (nchunk - 2, nchunk - 1):
      b = j % _NB
      pltpu.make_async_copy(rows[b], acc.at[didx.at[j]], ssem[b]).wait()
    plsc.subcore_barrier()
    pltpu.sync_copy(acc.at[pl.ds(r0, _RPT)], out.at[c, pl.ds(r0, _RPT)])

  return segsum


_SEGSUM128 = _make_segsum(_DI, _CH1, _NCHUNK1)
_SEGSUM16 = _make_segsum(16, _CH2, _NCHUNK2)

_BM = 1000


def _dense_mid(parts, xf, w1r, b1, w1o, w2r, w2o, b2):
  def body(p0, p1, xb, w1r_r, b1_r, w1o_r, w2r_r, w2o_r, b2_r, p2_o, r2_o):
    agg = p0[0] + p1[0]
    h = jnp.dot(agg, w1r_r[...], preferred_element_type=jnp.float32)
    h += b1_r[...]
    h += jnp.dot(xb[...], w1o_r[...], preferred_element_type=jnp.float32)
    h = jnp.maximum(h, 0.0)
    p2_o[...] = jnp.dot(h, w2r_r[...], preferred_element_type=jnp.float32)
    r2_o[...] = jnp.dot(h, w2o_r[...], preferred_element_type=jnp.float32) + b2_r[...]

  return pl.pallas_call(
      body,
      grid=(_N // _BM,),
      in_specs=[
          pl.BlockSpec((1, _BM, _DI), lambda i: (0, i, 0)),
          pl.BlockSpec((1, _BM, _DI), lambda i: (1, i, 0)),
          pl.BlockSpec((_BM, _DI), lambda i: (i, 0)),
          pl.BlockSpec((_DI, _DI), lambda i: (0, 0)),
          pl.BlockSpec((1, _DI), lambda i: (0, 0)),
          pl.BlockSpec((_DI, _DI), lambda i: (0, 0)),
          pl.BlockSpec((_DI, 16), lambda i: (0, 0)),
          pl.BlockSpec((_DI, 16), lambda i: (0, 0)),
          pl.BlockSpec((1, 16), lambda i: (0, 0)),
      ],
      out_specs=[
          pl.BlockSpec((_BM, 16), lambda i: (i, 0)),
          pl.BlockSpec((_BM, 16), lambda i: (i, 0)),
      ],
      out_shape=[
          jax.ShapeDtypeStruct((_N, 16), jnp.float32),
          jax.ShapeDtypeStruct((_N, 16), jnp.float32),
      ],
  )(parts, parts, xf, w1r, b1, w1o, w2r, w2o, b2)


def _final_add(parts2):
  def body(q0, q1, o):
    o[...] = q0[0] + q1[0]

  return pl.pallas_call(
      body,
      grid=(_N // _BM,),
      in_specs=[
          pl.BlockSpec((1, _BM, 16), lambda i: (0, i, 0)),
          pl.BlockSpec((1, _BM, 16), lambda i: (1, i, 0)),
      ],
      out_specs=pl.BlockSpec((_BM, 16), lambda i: (i, 0)),
      out_shape=jax.ShapeDtypeStruct((_N, 16), jnp.float32),
  )(parts2, parts2)


def _pack_edges(e, fill, ch, nchunk):
  ept = ch * nchunk
  return jnp.pad(e, (0, _NTILES * ept - _E), constant_values=fill).reshape(
      _NTILES, nchunk, ch)


def kernel(adj_est, x, W1_rel, b1_rel, W1_root, W2_rel, b2_rel, W2_root):
  xf = x.reshape(_N, _DI)
  src1 = _pack_edges(adj_est[0], 0, _CH1, _NCHUNK1)
  dst1 = _pack_edges(adj_est[1], _N, _CH1, _NCHUNK1)
  src2 = _pack_edges(adj_est[0], 0, _CH2, _NCHUNK2)
  dst2 = _pack_edges(adj_est[1], _N, _CH2, _NCHUNK2)

  init1 = jnp.zeros((_NSC, _NROWS, _DI), jnp.float32)
  parts1 = _SEGSUM128(xf, src1, dst1, init1)

  w2r = jnp.pad(W2_rel, ((0, 0), (0, 16 - W2_rel.shape[1])))
  w2o = jnp.pad(W2_root, ((0, 0), (0, 16 - W2_root.shape[1])))
  b2 = jnp.pad(b2_rel, (0, 16 - b2_rel.shape[0])).reshape(1, 16)
  p2, r2 = _dense_mid(parts1, xf, W1_rel, b1_rel.reshape(1, _DI), W1_root,
                      w2r, w2o, b2)

  init2 = jnp.zeros((_NSC, _NROWS, 16), jnp.float32).at[0, :_N, :].set(r2)
  parts2 = _SEGSUM16(p2, src2, dst2, init2)

  out16 = _final_add(parts2)
  return out16[:, :7].reshape(1, _N, 7)


# speedup vs baseline: 10.2967x; 1.0569x over previous
import functools

import jax
import jax.numpy as jnp
from jax import lax
from jax.experimental import pallas as pl
from jax.experimental.pallas import tpu as pltpu
from jax.experimental.pallas import tpu_sc as plsc

_N = 10000
_DI = 128
_E = 320000
_NSC = 2
_NSUB = 16
_NTILES = _NSC * _NSUB
_NROWS = 10016
_RPT = _NROWS // _NSUB
_CH1, _NCHUNK1 = 56, 180
_CH2, _NCHUNK2 = 128, 80
_NB = 4


def _make_segsum(d, ch, nchunk, seeded):
  mesh = plsc.VectorSubcoreMesh(core_axis_name="c", subcore_axis_name="s")
  nz = _RPT // ch
  rz = _RPT - nz * ch

  @functools.partial(
      pl.kernel,
      mesh=mesh,
      compiler_params=pltpu.CompilerParams(use_tc_tiling_on_sc=False),
      out_type=jax.ShapeDtypeStruct((_NSC, _NROWS, d), jnp.float32),
      scratch_types=[
          pltpu.VMEM((2, nchunk, ch), jnp.int32),
          [pltpu.VMEM((ch, d), jnp.float32)] * _NB,
          pltpu.VMEM_SHARED((_NROWS, d), jnp.float32),
          [pltpu.SemaphoreType.DMA] * _NB,
          [pltpu.SemaphoreType.DMA] * _NB,
          pltpu.SemaphoreType.DMA,
      ],
  )
  def segsum(table, edges, seed, out, idx, rows, acc, gsem, ssem, zsem):
    c = lax.axis_index("c")
    s = lax.axis_index("s")
    wid = c * _NSUB + s
    r0 = s * _RPT
    sidx = idx.at[0]
    didx = idx.at[1]
    pltpu.sync_copy(edges.at[0, wid], sidx)
    pltpu.sync_copy(edges.at[1, wid], didx)
    for b in range(2):
      pltpu.async_copy(table.at[sidx.at[b]], rows[b], gsem[b])
    if seeded:
      @pl.when(c == 0)
      def _():
        pltpu.sync_copy(seed.at[pl.ds(r0, _RPT)], acc.at[pl.ds(r0, _RPT)])

    @pl.when((c != 0) if seeded else (c == c))
    def _():
      z16 = jnp.zeros((16,), jnp.float32)

      def zrow(i, carry):
        for k in range(d // 16):
          rows[2][i, pl.ds(16 * k, 16)] = z16
        return carry

      lax.fori_loop(0, ch, zrow, 0)
      for q in range(nz):
        pltpu.async_copy(rows[2], acc.at[pl.ds(r0 + q * ch, ch)], zsem)
      if rz:
        pltpu.async_copy(
            rows[2].at[pl.ds(0, rz)], acc.at[pl.ds(r0 + nz * ch, rz)], zsem)
      for q in range(nz):
        pltpu.make_async_copy(rows[2], acc.at[pl.ds(r0 + q * ch, ch)],
                              zsem).wait()
      if rz:
        pltpu.make_async_copy(
            rows[2].at[pl.ds(0, rz)], acc.at[pl.ds(r0 + nz * ch, rz)],
            zsem).wait()

    plsc.subcore_barrier()

    def grp(k, carry):
      j0 = _NB * k
      for b in range(_NB):
        j = j0 + b
        pltpu.make_async_copy(table.at[sidx.at[j]], rows[b], gsem[b]).wait()
        pltpu.async_copy(rows[b], acc.at[didx.at[j]], ssem[b], add=True)
        b2 = (b + 2) % _NB

        @pl.when(j >= 2)
        def _():
          pltpu.make_async_copy(
              rows[b2], acc.at[didx.at[j - 2]], ssem[b2]).wait()

        @pl.when(j + 2 < nchunk)
        def _():
          pltpu.async_copy(table.at[sidx.at[j + 2]], rows[b2], gsem[b2])
      return carry

    lax.fori_loop(0, nchunk // _NB, grp, 0)
    for j in (nchunk - 2, nchunk - 1):
      b = j % _NB
      pltpu.make_async_copy(rows[b], acc.at[didx.at[j]], ssem[b]).wait()
    plsc.subcore_barrier()
    pltpu.sync_copy(acc.at[pl.ds(r0, _RPT)], out.at[c, pl.ds(r0, _RPT)])

  return segsum


_SEGSUM128 = _make_segsum(_DI, _CH1, _NCHUNK1, seeded=False)
_SEGSUM16 = _make_segsum(16, _CH2, _NCHUNK2, seeded=True)

_BM = 2504


def _dense_mid(parts, xf, w1r, b1, w1o, w2r, w2o, b2):
  def body(p0, p1, xb, w1r_r, b1_r, w1o_r, w2r_r, w2o_r, b2_r, p2_o, r2_o):
    agg = p0[0] + p1[0]
    h = jnp.dot(agg, w1r_r[...], preferred_element_type=jnp.float32)
    h += b1_r[...]
    h += jnp.dot(xb[...], w1o_r[...], preferred_element_type=jnp.float32)
    h = jnp.maximum(h, 0.0)
    p2_o[...] = jnp.dot(h, w2r_r[...], preferred_element_type=jnp.float32)
    r2_o[...] = jnp.dot(h, w2o_r[...], preferred_element_type=jnp.float32) + b2_r[...]

  return pl.pallas_call(
      body,
      grid=(_NROWS // _BM,),
      in_specs=[
          pl.BlockSpec((1, _BM, _DI), lambda i: (0, i, 0)),
          pl.BlockSpec((1, _BM, _DI), lambda i: (1, i, 0)),
          pl.BlockSpec((_BM, _DI), lambda i: (i, 0)),
          pl.BlockSpec((_DI, _DI), lambda i: (0, 0)),
          pl.BlockSpec((1, _DI), lambda i: (0, 0)),
          pl.BlockSpec((_DI, _DI), lambda i: (0, 0)),
          pl.BlockSpec((_DI, 16), lambda i: (0, 0)),
          pl.BlockSpec((_DI, 16), lambda i: (0, 0)),
          pl.BlockSpec((1, 16), lambda i: (0, 0)),
      ],
      out_specs=[
          pl.BlockSpec((_BM, 16), lambda i: (i, 0)),
          pl.BlockSpec((_BM, 16), lambda i: (i, 0)),
      ],
      out_shape=[
          jax.ShapeDtypeStruct((_NROWS, 16), jnp.float32),
          jax.ShapeDtypeStruct((_NROWS, 16), jnp.float32),
      ],
  )(parts, parts, xf, w1r, b1, w1o, w2r, w2o, b2)


def _final_add(parts2):
  def body(q0, q1, o):
    o[...] = q0[0] + q1[0]

  return pl.pallas_call(
      body,
      grid=(_NROWS // _BM,),
      in_specs=[
          pl.BlockSpec((1, _BM, 16), lambda i: (0, i, 0)),
          pl.BlockSpec((1, _BM, 16), lambda i: (1, i, 0)),
      ],
      out_specs=pl.BlockSpec((_BM, 16), lambda i: (i, 0)),
      out_shape=jax.ShapeDtypeStruct((_N, 16), jnp.float32),
  )(parts2, parts2)


def _pack_edges(adj, ch, nchunk):
  ept = ch * nchunk
  npad = _NTILES * ept - _E
  fill = jnp.stack([
      jnp.zeros((npad,), jnp.int32),
      _N + (jnp.arange(npad, dtype=jnp.int32) % (_NROWS - _N)),
  ])
  return jnp.concatenate([adj, fill], axis=1).reshape(2, _NTILES, nchunk, ch)


def kernel(adj_est, x, W1_rel, b1_rel, W1_root, W2_rel, b2_rel, W2_root):
  xf = x.reshape(_N, _DI)
  edges1 = _pack_edges(adj_est, _CH1, _NCHUNK1)
  edges2 = _pack_edges(adj_est, _CH2, _NCHUNK2)

  parts1 = _SEGSUM128(xf, edges1, xf)

  w2r = jnp.pad(W2_rel, ((0, 0), (0, 16 - W2_rel.shape[1])))
  w2o = jnp.pad(W2_root, ((0, 0), (0, 16 - W2_root.shape[1])))
  b2 = jnp.pad(b2_rel, (0, 16 - b2_rel.shape[0])).reshape(1, 16)
  p2, r2 = _dense_mid(parts1, xf, W1_rel, b1_rel.reshape(1, _DI), W1_root,
                      w2r, w2o, b2)

  parts2 = _SEGSUM16(p2, edges2, r2)

  out16 = _final_add(parts2)
  return out16[:, :7].reshape(1, _N, 7)
